# baseline (speedup 1.0000x reference)
import functools

import jax
import jax.numpy as jnp
from jax import lax
from jax.experimental import pallas as pl
from jax.experimental.pallas import tpu as pltpu


def _conv_body(xt_ref, w_ref, b_ref, o_ref, *, H, W, kh, kw, n_ext, wo):
    pieces = [pltpu.bitcast(xt_ref[0, min(h, H - 1)], jnp.int32)
              for h in range(H + n_ext)]
    slab = jnp.concatenate(pieces, axis=1)
    parts = [
        slab[:, dh * W + dw:dh * W + dw + H * W]
        for dh in range(kh)
        for dw in range(kw)
    ]
    xk = pltpu.bitcast(jnp.concatenate(parts, axis=0),
                       jnp.bfloat16)
    acc = lax.dot_general(
        w_ref[...], xk, (((1,), (0,)), ((), ())),
        preferred_element_type=jnp.float32)
    acc = acc + b_ref[:, :1]
    ho = o_ref.shape[2] // wo
    o_ref[0] = jnp.concatenate(
        [acc[:, h * W:h * W + wo] for h in range(ho)], axis=1)


@jax.jit
def _conv2d(x, w, b):
    C_out, C_in, kh, kw = w.shape
    B, _, H, W = x.shape
    Ho = H - kh + 1
    Wo = W - kw + 1
    P = H * W
    n_ext = kh

    xt = x.transpose(0, 2, 1, 3).astype(jnp.bfloat16)
    wp = w.transpose(0, 2, 3, 1).reshape(C_out, kh * kw * C_in)
    wp = wp.astype(jnp.bfloat16)
    bb = jnp.broadcast_to(b.astype(jnp.float32).reshape(C_out, 1),
                          (C_out, 128))

    body = functools.partial(_conv_body, H=H, W=W, kh=kh, kw=kw, n_ext=n_ext,
                             wo=Wo)
    y = pl.pallas_call(
        body,
        out_shape=jax.ShapeDtypeStruct((B, C_out, Ho * Wo), jnp.float32),
        grid=(B,),
        in_specs=[
            pl.BlockSpec((1, H, C_in, W), lambda bi: (bi, 0, 0, 0)),
            pl.BlockSpec((C_out, kh * kw * C_in), lambda bi: (0, 0)),
            pl.BlockSpec((C_out, 128), lambda bi: (0, 0)),
        ],
        out_specs=pl.BlockSpec((1, C_out, Ho * Wo), lambda bi: (bi, 0, 0)),
        compiler_params=pltpu.CompilerParams(
            dimension_semantics=("parallel",),
            vmem_limit_bytes=int(48 << 20)),
    )(xt, wp, bb)

    return y.reshape(B, C_out, Ho, Wo)


def kernel(x, w, b):
    return _conv2d(x, w, b)


# speedup vs baseline: 1.1122x; 1.1122x over previous
import functools

import jax
import jax.numpy as jnp
from jax import lax
from jax.experimental import pallas as pl
from jax.experimental.pallas import tpu as pltpu


def _conv_body(xt_ref, w_ref, b_ref, o_ref, *, H, W, kh, kw, n_ext, wo, np_c):
    pieces = [pltpu.bitcast(xt_ref[0, min(h, H - 1)], jnp.int32)
              for h in range(H + n_ext)]
    slab = jnp.concatenate(pieces, axis=1)
    parts = [
        slab[:, dh * W + dw:dh * W + dw + np_c]
        for dh in range(kh)
        for dw in range(kw)
    ]
    xk = pltpu.bitcast(jnp.concatenate(parts, axis=0),
                       jnp.bfloat16)
    acc = lax.dot_general(
        w_ref[...], xk, (((1,), (0,)), ((), ())),
        preferred_element_type=jnp.float32)
    acc = acc + b_ref[:, :1]
    ho = o_ref.shape[2] // wo
    o_ref[0] = jnp.concatenate(
        [acc[:, h * W:h * W + wo] for h in range(ho)],
        axis=1).astype(o_ref.dtype)


@jax.jit
def _conv2d(x, w, b):
    C_out, C_in, kh, kw = w.shape
    B, _, H, W = x.shape
    Ho = H - kh + 1
    Wo = W - kw + 1
    P = H * W
    n_ext = kh

    xt = x.transpose(0, 2, 1, 3).astype(jnp.bfloat16)
    wp = w.transpose(0, 2, 3, 1).reshape(C_out, kh * kw * C_in)
    wp = wp.astype(jnp.bfloat16)
    bb = jnp.broadcast_to(b.astype(jnp.float32).reshape(C_out, 1),
                          (C_out, 128))

    np_c = pl.cdiv((Ho - 1) * W + Wo, 128) * 128
    body = functools.partial(_conv_body, H=H, W=W, kh=kh, kw=kw, n_ext=n_ext,
                             wo=Wo, np_c=np_c)
    y = pl.pallas_call(
        body,
        out_shape=jax.ShapeDtypeStruct((B, C_out, Ho * Wo), jnp.bfloat16),
        grid=(B,),
        in_specs=[
            pl.BlockSpec((1, H, C_in, W), lambda bi: (bi, 0, 0, 0)),
            pl.BlockSpec((C_out, kh * kw * C_in), lambda bi: (0, 0)),
            pl.BlockSpec((C_out, 128), lambda bi: (0, 0)),
        ],
        out_specs=pl.BlockSpec((1, C_out, Ho * Wo), lambda bi: (bi, 0, 0)),
        compiler_params=pltpu.CompilerParams(
            dimension_semantics=("parallel",),
            vmem_limit_bytes=int(48 << 20)),
    )(xt, wp, bb)

    return y.reshape(B, C_out, Ho, Wo).astype(jnp.float32)


def kernel(x, w, b):
    return _conv2d(x, w, b)


# speedup vs baseline: 1.1160x; 1.0034x over previous
import functools

import jax
import jax.numpy as jnp
from jax import lax
from jax.experimental import pallas as pl
from jax.experimental.pallas import tpu as pltpu


def _conv_body(xt_ref, w_ref, b_ref, o_ref, *, H, W, kh, kw, n_ext, wo, np_c):
    pieces = [pltpu.bitcast(xt_ref[0, min(h, H - 1)], jnp.int32)
              for h in range(H + n_ext)]
    slab = jnp.concatenate(pieces, axis=1)
    ext_np = np_c + (kh - 1) * W
    shifted = [slab[:, dw:dw + ext_np] for dw in range(kw)]
    parts = [
        shifted[dw][:, dh * W:dh * W + np_c]
        for dh in range(kh)
        for dw in range(kw)
    ]
    xk = pltpu.bitcast(jnp.concatenate(parts, axis=0),
                       jnp.bfloat16)
    acc = lax.dot_general(
        w_ref[...], xk, (((1,), (0,)), ((), ())),
        preferred_element_type=jnp.float32)
    acc = acc + b_ref[:, :1]
    ho = o_ref.shape[2] // wo
    o_ref[0] = jnp.concatenate(
        [acc[:, h * W:h * W + wo] for h in range(ho)],
        axis=1).astype(o_ref.dtype)


@jax.jit
def _conv2d(x, w, b):
    C_out, C_in, kh, kw = w.shape
    B, _, H, W = x.shape
    Ho = H - kh + 1
    Wo = W - kw + 1
    P = H * W
    n_ext = kh

    xt = x.transpose(0, 2, 1, 3).astype(jnp.bfloat16)
    wp = w.transpose(0, 2, 3, 1).reshape(C_out, kh * kw * C_in)
    wp = wp.astype(jnp.bfloat16)
    bb = jnp.broadcast_to(b.astype(jnp.float32).reshape(C_out, 1),
                          (C_out, 128))

    np_c = pl.cdiv((Ho - 1) * W + Wo, 128) * 128
    body = functools.partial(_conv_body, H=H, W=W, kh=kh, kw=kw, n_ext=n_ext,
                             wo=Wo, np_c=np_c)
    y = pl.pallas_call(
        body,
        out_shape=jax.ShapeDtypeStruct((B, C_out, Ho * Wo), jnp.bfloat16),
        grid=(B,),
        in_specs=[
            pl.BlockSpec((1, H, C_in, W), lambda bi: (bi, 0, 0, 0)),
            pl.BlockSpec((C_out, kh * kw * C_in), lambda bi: (0, 0)),
            pl.BlockSpec((C_out, 128), lambda bi: (0, 0)),
        ],
        out_specs=pl.BlockSpec((1, C_out, Ho * Wo), lambda bi: (bi, 0, 0)),
        compiler_params=pltpu.CompilerParams(
            dimension_semantics=("parallel",),
            vmem_limit_bytes=int(48 << 20)),
    )(xt, wp, bb)

    return y.reshape(B, C_out, Ho, Wo).astype(jnp.float32)


def kernel(x, w, b):
    return _conv2d(x, w, b)
